# baseline (speedup 1.0000x reference)
import functools

import jax
import jax.numpy as jnp
from jax import lax
from jax.experimental import pallas as pl
from jax.experimental.pallas import tpu as pltpu
from jax.experimental.pallas import tpu_sc as plsc

_B, _S, _D = 16, 2048, 768
_W = 1024
_NL = 9
_PAD = 16

_NC, _NS = 2, 16
_CHUNK = 1
_CB = _B // _CHUNK
_QP = _NC * _NS // _CB
_SS = _S // _QP


_RB = 4


def _tc1_body(x_ref, w_ref, p_ref):
    x = x_ref[...].reshape(_RB * _S, _D)
    w = w_ref[...]
    logits = jnp.dot(x, w, preferred_element_type=jnp.float32)
    m = jnp.max(logits, axis=-1, keepdims=True)
    e = jnp.exp(logits - m)
    p = e / jnp.sum(e, axis=-1, keepdims=True)
    p16 = jnp.concatenate(
        [
            p,
            jnp.ones((_RB * _S, 1), jnp.float32),
            jnp.zeros((_RB * _S, _PAD - _NL - 1), jnp.float32),
        ],
        axis=1,
    )
    p_ref[...] = jnp.swapaxes(p16.reshape(_RB, _S, _PAD), 1, 2)


def _tc1_call(x, w, ci):
    return pl.pallas_call(
        _tc1_body,
        grid=(_CB // _RB,),
        in_specs=[
            pl.BlockSpec((_RB, _S, _D), lambda i: (ci * _CB // _RB + i, 0, 0)),
            pl.BlockSpec((_D, _NL), lambda i: (0, 0)),
        ],
        out_specs=pl.BlockSpec((_RB, _PAD, _S), lambda i: (i, 0, 0)),
        out_shape=jax.ShapeDtypeStruct((_CB, _PAD, _S), jnp.float32),
        compiler_params=pltpu.CompilerParams(vmem_limit_bytes=100 * 1024 * 1024),
    )(x, w)


def _sc_seg_sum(probs_t, word_ids, zeros_hbm_arr, ci):
    mesh = plsc.VectorSubcoreMesh(core_axis_name="c", subcore_axis_name="s")

    @functools.partial(
        pl.kernel,
        mesh=mesh,
        out_type=jax.ShapeDtypeStruct((_CB, _QP, _PAD, _W), jnp.float32),
        compiler_params=pltpu.CompilerParams(needs_layout_passes=False),
        scratch_types=[
            pltpu.VMEM((_SS,), jnp.int32),
            pltpu.VMEM((_PAD, _SS), jnp.float32),
            pltpu.VMEM((_PAD, _W), jnp.float32),
        ],
    )
    def k(probs_hbm, wid_hbm, zeros_hbm, out_hbm, idx_v, pv, acc):
        c = lax.axis_index("c")
        s = lax.axis_index("s")
        unit = c * _NS + s
        b = unit // _QP
        q = unit % _QP

        pltpu.sync_copy(wid_hbm.at[ci * _CB + b, pl.ds(q * _SS, _SS)], idx_v)
        pltpu.sync_copy(probs_hbm.at[b, :, pl.ds(q * _SS, _SS)], pv)
        pltpu.sync_copy(zeros_hbm, acc)

        def body(t, carry):
            sl = pl.ds(t * 16, 16)
            idx = idx_v[sl]
            for j in range(_NL + 1):
                jrow = jnp.full((16,), j, jnp.int32)
                plsc.addupdate_scatter(acc, [jrow, idx], pv[j, sl])
            return carry

        lax.fori_loop(0, _SS // 16, body, 0)
        pltpu.sync_copy(acc, out_hbm.at[b, q])

    return k(probs_t, word_ids, zeros_hbm_arr)


def _tc2_body(*refs):
    s_refs = refs[:_CHUNK]
    lab_ref, avg_ref, loss_ref = refs[_CHUNK:]
    chunks = []
    for r in s_refs:
        part = r[...]
        acc = part[:, 0]
        for q in range(1, _QP):
            acc = acc + part[:, q]
        chunks.append(acc)
    data = jnp.concatenate(chunks, axis=0)
    cnt = data[:, _NL:_NL + 1, :]
    avg = data / jnp.maximum(cnt, 1.0)
    row = lax.broadcasted_iota(jnp.int32, avg.shape, 1)
    ml = jnp.where(row < _NL, avg, -1e30)
    m = jnp.max(ml, axis=1, keepdims=True)
    se = jnp.sum(jnp.exp(ml - m), axis=1, keepdims=True)
    lab = lab_ref[...]
    picked = jnp.sum(jnp.where(row == lab, avg, 0.0), axis=1, keepdims=True)
    nll = m + jnp.log(se) - picked
    avg_ref[...] = jnp.swapaxes(avg, 1, 2)
    loss_ref[0, 0] = jnp.sum(nll) * (1.0 / (_B * _W))


def kernel(sent_logits, word_ids, labels, W_mlp):
    zeros_arr = jnp.zeros((_PAD, _W), jnp.float32)
    sums = []
    for ci in range(_CHUNK):
        probs_c = _tc1_call(sent_logits, W_mlp, ci)
        sums.append(_sc_seg_sum(probs_c, word_ids, zeros_arr, ci))

    avg, loss = pl.pallas_call(
        _tc2_body,
        in_specs=[
            pl.BlockSpec((_CB, _QP, _PAD, _W), lambda: (0, 0, 0, 0))
            for _ in range(_CHUNK)
        ] + [pl.BlockSpec((_B, 1, _W), lambda: (0, 0, 0))],
        out_specs=[
            pl.BlockSpec((_B, _W, _PAD), lambda: (0, 0, 0)),
            pl.BlockSpec((1, 1), lambda: (0, 0), memory_space=pltpu.SMEM),
        ],
        out_shape=[
            jax.ShapeDtypeStruct((_B, _W, _PAD), jnp.float32),
            jax.ShapeDtypeStruct((1, 1), jnp.float32),
        ],
    )(*sums, labels.reshape(_B, 1, _W))

    return avg[:, :, :_NL], loss[0, 0]


# speedup vs baseline: 1.0146x; 1.0146x over previous
import functools

import jax
import jax.numpy as jnp
from jax import lax
from jax.experimental import pallas as pl
from jax.experimental.pallas import tpu as pltpu
from jax.experimental.pallas import tpu_sc as plsc

_B, _S, _D = 16, 2048, 768
_W = 1024
_NL = 9
_PAD = 16

_NC, _NS = 2, 16
_CHUNK = 1
_CB = _B // _CHUNK
_QP = _NC * _NS // _CB
_SS = _S // _QP


_RB = 2


def _tc1_body(x_ref, w_ref, p_ref):
    x = x_ref[...].reshape(_RB * _S, _D)
    w = w_ref[...]
    logits = jnp.dot(x, w, preferred_element_type=jnp.float32)
    m = jnp.max(logits, axis=-1, keepdims=True)
    e = jnp.exp(logits - m)
    p = e / jnp.sum(e, axis=-1, keepdims=True)
    p16 = jnp.concatenate(
        [
            p,
            jnp.ones((_RB * _S, 1), jnp.float32),
            jnp.zeros((_RB * _S, _PAD - _NL - 1), jnp.float32),
        ],
        axis=1,
    )
    p_ref[...] = jnp.swapaxes(p16.reshape(_RB, _S, _PAD), 1, 2)


def _tc1_call(x, w, ci):
    return pl.pallas_call(
        _tc1_body,
        grid=(_CB // _RB,),
        in_specs=[
            pl.BlockSpec((_RB, _S, _D), lambda i: (ci * _CB // _RB + i, 0, 0)),
            pl.BlockSpec((_D, _NL), lambda i: (0, 0)),
        ],
        out_specs=pl.BlockSpec((_RB, _PAD, _S), lambda i: (i, 0, 0)),
        out_shape=jax.ShapeDtypeStruct((_CB, _PAD, _S), jnp.float32),
        compiler_params=pltpu.CompilerParams(vmem_limit_bytes=100 * 1024 * 1024),
    )(x, w)


def _sc_seg_sum(probs_t, word_ids, zeros_hbm_arr, ci):
    mesh = plsc.VectorSubcoreMesh(core_axis_name="c", subcore_axis_name="s")

    @functools.partial(
        pl.kernel,
        mesh=mesh,
        out_type=jax.ShapeDtypeStruct((_CB, _QP, _PAD, _W), jnp.float32),
        compiler_params=pltpu.CompilerParams(needs_layout_passes=False),
        scratch_types=[
            pltpu.VMEM((_SS,), jnp.int32),
            pltpu.VMEM((_PAD, _SS), jnp.float32),
            pltpu.VMEM((_PAD, _W), jnp.float32),
        ],
    )
    def k(probs_hbm, wid_hbm, zeros_hbm, out_hbm, idx_v, pv, acc):
        c = lax.axis_index("c")
        s = lax.axis_index("s")
        unit = c * _NS + s
        b = unit // _QP
        q = unit % _QP

        pltpu.sync_copy(wid_hbm.at[ci * _CB + b, pl.ds(q * _SS, _SS)], idx_v)
        pltpu.sync_copy(probs_hbm.at[b, :, pl.ds(q * _SS, _SS)], pv)
        pltpu.sync_copy(zeros_hbm, acc)

        def body(t, carry):
            sl = pl.ds(t * 16, 16)
            idx = idx_v[sl]
            for j in range(_NL + 1):
                jrow = jnp.full((16,), j, jnp.int32)
                plsc.addupdate_scatter(acc, [jrow, idx], pv[j, sl])
            return carry

        lax.fori_loop(0, _SS // 16, body, 0)
        pltpu.sync_copy(acc, out_hbm.at[b, q])

    return k(probs_t, word_ids, zeros_hbm_arr)


def _tc2_body(*refs):
    s_refs = refs[:_CHUNK]
    lab_ref, avg_ref, loss_ref = refs[_CHUNK:]
    chunks = []
    for r in s_refs:
        part = r[...]
        acc = part[:, 0]
        for q in range(1, _QP):
            acc = acc + part[:, q]
        chunks.append(acc)
    data = jnp.concatenate(chunks, axis=0)
    cnt = data[:, _NL:_NL + 1, :]
    avg = data / jnp.maximum(cnt, 1.0)
    row = lax.broadcasted_iota(jnp.int32, avg.shape, 1)
    ml = jnp.where(row < _NL, avg, -1e30)
    m = jnp.max(ml, axis=1, keepdims=True)
    se = jnp.sum(jnp.exp(ml - m), axis=1, keepdims=True)
    lab = lab_ref[...]
    picked = jnp.sum(jnp.where(row == lab, avg, 0.0), axis=1, keepdims=True)
    nll = m + jnp.log(se) - picked
    avg_ref[...] = jnp.swapaxes(avg, 1, 2)
    loss_ref[0, 0] = jnp.sum(nll) * (1.0 / (_B * _W))


def kernel(sent_logits, word_ids, labels, W_mlp):
    zeros_arr = jnp.zeros((_PAD, _W), jnp.float32)
    sums = []
    for ci in range(_CHUNK):
        probs_c = _tc1_call(sent_logits, W_mlp, ci)
        sums.append(_sc_seg_sum(probs_c, word_ids, zeros_arr, ci))

    avg, loss = pl.pallas_call(
        _tc2_body,
        in_specs=[
            pl.BlockSpec((_CB, _QP, _PAD, _W), lambda: (0, 0, 0, 0))
            for _ in range(_CHUNK)
        ] + [pl.BlockSpec((_B, 1, _W), lambda: (0, 0, 0))],
        out_specs=[
            pl.BlockSpec((_B, _W, _PAD), lambda: (0, 0, 0)),
            pl.BlockSpec((1, 1), lambda: (0, 0), memory_space=pltpu.SMEM),
        ],
        out_shape=[
            jax.ShapeDtypeStruct((_B, _W, _PAD), jnp.float32),
            jax.ShapeDtypeStruct((1, 1), jnp.float32),
        ],
    )(*sums, labels.reshape(_B, 1, _W))

    return avg[:, :, :_NL], loss[0, 0]


# speedup vs baseline: 1.1215x; 1.1053x over previous
import functools

import jax
import jax.numpy as jnp
from jax import lax
from jax.experimental import pallas as pl
from jax.experimental.pallas import tpu as pltpu
from jax.experimental.pallas import tpu_sc as plsc

_B, _S, _D = 16, 2048, 768
_W = 1024
_NL = 9
_PAD = 16

_NC, _NS = 2, 16
_CHUNK = 1
_CB = _B // _CHUNK
_QP = _NC * _NS // _CB
_SS = _S // _QP


_RB = 2


def _tc1_body(x_ref, w_ref, p_ref):
    x = x_ref[...].reshape(_RB * _S, _D)
    w = w_ref[...]
    logits = jnp.dot(x, w, preferred_element_type=jnp.float32)
    m = jnp.max(logits, axis=-1, keepdims=True)
    e = jnp.exp(logits - m)
    p = e / jnp.sum(e, axis=-1, keepdims=True)
    p16 = jnp.concatenate(
        [
            p,
            jnp.ones((_RB * _S, 1), jnp.float32),
            jnp.zeros((_RB * _S, _PAD - _NL - 1), jnp.float32),
        ],
        axis=1,
    )
    p_ref[...] = jnp.swapaxes(p16.reshape(_RB, _S, _PAD), 1, 2)


def _tc1_call(x, w, ci):
    return pl.pallas_call(
        _tc1_body,
        grid=(_CB // _RB,),
        in_specs=[
            pl.BlockSpec((_RB, _S, _D), lambda i: (ci * _CB // _RB + i, 0, 0)),
            pl.BlockSpec((_D, _NL), lambda i: (0, 0)),
        ],
        out_specs=pl.BlockSpec((_RB, _PAD, _S), lambda i: (i, 0, 0)),
        out_shape=jax.ShapeDtypeStruct((_CB, _PAD, _S), jnp.float32),
        compiler_params=pltpu.CompilerParams(vmem_limit_bytes=100 * 1024 * 1024),
    )(x, w)


def _sc_seg_sum(probs_t, word_ids, zeros_hbm_arr, ci):
    mesh = plsc.VectorSubcoreMesh(core_axis_name="c", subcore_axis_name="s")

    @functools.partial(
        pl.kernel,
        mesh=mesh,
        out_type=jax.ShapeDtypeStruct((_CB, _QP, _PAD, _W), jnp.float32),
        compiler_params=pltpu.CompilerParams(needs_layout_passes=False),
        scratch_types=[
            pltpu.VMEM((_SS,), jnp.int32),
            pltpu.VMEM((_PAD, _SS), jnp.float32),
            pltpu.VMEM((_PAD, _W), jnp.float32),
        ],
    )
    def k(probs_hbm, wid_hbm, zeros_hbm, out_hbm, idx_v, pv, acc):
        c = lax.axis_index("c")
        s = lax.axis_index("s")
        unit = c * _NS + s
        b = unit // _QP
        q = unit % _QP

        pltpu.sync_copy(wid_hbm.at[ci * _CB + b, pl.ds(q * _SS, _SS)], idx_v)
        pltpu.sync_copy(probs_hbm.at[b, :, pl.ds(q * _SS, _SS)], pv)
        pltpu.sync_copy(zeros_hbm, acc)

        def body(t, carry):
            sl = pl.ds(t * 16, 16)
            idx = idx_v[sl]
            for j in range(_NL + 1):
                jrow = jnp.full((16,), j, jnp.int32)
                plsc.addupdate_scatter(acc, [jrow, idx], pv[j, sl])
            return carry

        lax.fori_loop(0, _SS // 16, body, 0)
        pltpu.sync_copy(acc, out_hbm.at[b, q])

    return k(probs_t, word_ids, zeros_hbm_arr)


def _tc2_body(*refs):
    s_refs = refs[:_CHUNK]
    lab_ref, avg_ref, loss_ref = refs[_CHUNK:]
    chunks = []
    for r in s_refs:
        part = r[...]
        acc = part[:, 0]
        for q in range(1, _QP):
            acc = acc + part[:, q]
        chunks.append(acc)
    data = jnp.concatenate(chunks, axis=0)
    cnt = data[:, _NL:_NL + 1, :]
    avg = data / jnp.maximum(cnt, 1.0)
    row = lax.broadcasted_iota(jnp.int32, avg.shape, 1)
    ml = jnp.where(row < _NL, avg, -1e30)
    m = jnp.max(ml, axis=1, keepdims=True)
    se = jnp.sum(jnp.exp(ml - m), axis=1, keepdims=True)
    lab = lab_ref[...]
    picked = jnp.sum(jnp.where(row == lab, avg, 0.0), axis=1, keepdims=True)
    nll = m + jnp.log(se) - picked
    avg_ref[...] = avg
    loss_ref[0, 0] = jnp.sum(nll) * (1.0 / (_B * _W))


def kernel(sent_logits, word_ids, labels, W_mlp):
    zeros_arr = jnp.zeros((_PAD, _W), jnp.float32)
    sums = []
    for ci in range(_CHUNK):
        probs_c = _tc1_call(sent_logits, W_mlp, ci)
        sums.append(_sc_seg_sum(probs_c, word_ids, zeros_arr, ci))

    avg, loss = pl.pallas_call(
        _tc2_body,
        in_specs=[
            pl.BlockSpec((_CB, _QP, _PAD, _W), lambda: (0, 0, 0, 0))
            for _ in range(_CHUNK)
        ] + [pl.BlockSpec((_B, 1, _W), lambda: (0, 0, 0))],
        out_specs=[
            pl.BlockSpec((_B, _PAD, _W), lambda: (0, 0, 0)),
            pl.BlockSpec((1, 1), lambda: (0, 0), memory_space=pltpu.SMEM),
        ],
        out_shape=[
            jax.ShapeDtypeStruct((_B, _PAD, _W), jnp.float32),
            jax.ShapeDtypeStruct((1, 1), jnp.float32),
        ],
    )(*sums, labels.reshape(_B, 1, _W))

    return jnp.swapaxes(avg[:, :_NL, :], 1, 2), loss[0, 0]


# speedup vs baseline: 1.1266x; 1.0045x over previous
import functools

import jax
import jax.numpy as jnp
from jax import lax
from jax.experimental import pallas as pl
from jax.experimental.pallas import tpu as pltpu
from jax.experimental.pallas import tpu_sc as plsc

_B, _S, _D = 16, 2048, 768
_W = 1024
_NL = 9
_PAD = 16

_NC, _NS = 2, 16
_CHUNK = 1
_CB = _B // _CHUNK
_QP = _NC * _NS // _CB
_SS = _S // _QP


_RB = 2


def _tc1_body(x_ref, w_ref, p_ref):
    x = x_ref[...].reshape(_RB * _S, _D)
    w = w_ref[...]
    logits = jnp.dot(x, w, preferred_element_type=jnp.float32)
    m = jnp.max(logits, axis=-1, keepdims=True)
    e = jnp.exp(logits - m)
    p = e / jnp.sum(e, axis=-1, keepdims=True)
    p16 = jnp.concatenate(
        [
            p,
            jnp.ones((_RB * _S, 1), jnp.float32),
            jnp.zeros((_RB * _S, _PAD - _NL - 1), jnp.float32),
        ],
        axis=1,
    )
    p_ref[...] = jnp.swapaxes(p16.reshape(_RB, _S, _PAD), 1, 2)


def _tc1_call(x, w, ci):
    return pl.pallas_call(
        _tc1_body,
        grid=(_CB // _RB,),
        in_specs=[
            pl.BlockSpec((_RB, _S, _D), lambda i: (ci * _CB // _RB + i, 0, 0)),
            pl.BlockSpec((_D, _NL), lambda i: (0, 0)),
        ],
        out_specs=pl.BlockSpec((_RB, _PAD, _S), lambda i: (i, 0, 0)),
        out_shape=jax.ShapeDtypeStruct((_CB, _PAD, _S), jnp.float32),
        compiler_params=pltpu.CompilerParams(vmem_limit_bytes=100 * 1024 * 1024),
    )(x, w)


def _sc_seg_sum(probs_t, word_ids, zeros_hbm_arr, ci):
    mesh = plsc.VectorSubcoreMesh(core_axis_name="c", subcore_axis_name="s")

    @functools.partial(
        pl.kernel,
        mesh=mesh,
        out_type=jax.ShapeDtypeStruct((_CB, _QP, _PAD, _W), jnp.float32),
        compiler_params=pltpu.CompilerParams(needs_layout_passes=False),
        scratch_types=[
            pltpu.VMEM((_SS,), jnp.int32),
            pltpu.VMEM((_PAD, _SS), jnp.float32),
            pltpu.VMEM((_PAD, _W), jnp.float32),
            pltpu.SemaphoreType.DMA,
            pltpu.SemaphoreType.DMA,
            pltpu.SemaphoreType.DMA,
        ],
    )
    def k(probs_hbm, wid_hbm, zeros_hbm, out_hbm, idx_v, pv, acc, s0, s1, s2):
        c = lax.axis_index("c")
        s = lax.axis_index("s")
        unit = c * _NS + s
        b = unit // _QP
        q = unit % _QP

        c0 = pltpu.async_copy(
            wid_hbm.at[ci * _CB + b, pl.ds(q * _SS, _SS)], idx_v, s0)
        c1 = pltpu.async_copy(
            probs_hbm.at[b, :, pl.ds(q * _SS, _SS)], pv, s1)
        c2 = pltpu.async_copy(zeros_hbm, acc, s2)
        c0.wait()
        c1.wait()
        c2.wait()

        def body(t, carry):
            sl = pl.ds(t * 16, 16)
            idx = idx_v[sl]
            for j in range(_NL + 1):
                jrow = jnp.full((16,), j, jnp.int32)
                plsc.addupdate_scatter(acc, [jrow, idx], pv[j, sl])
            return carry

        lax.fori_loop(0, _SS // 16, body, 0)
        pltpu.sync_copy(acc, out_hbm.at[b, q])

    return k(probs_t, word_ids, zeros_hbm_arr)


def _tc2_body(*refs):
    s_refs = refs[:_CHUNK]
    lab_ref, avg_ref, loss_ref = refs[_CHUNK:]
    chunks = []
    for r in s_refs:
        part = r[...]
        acc = part[:, 0]
        for q in range(1, _QP):
            acc = acc + part[:, q]
        chunks.append(acc)
    data = jnp.concatenate(chunks, axis=0)
    cnt = data[:, _NL:_NL + 1, :]
    avg = data / jnp.maximum(cnt, 1.0)
    row = lax.broadcasted_iota(jnp.int32, avg.shape, 1)
    ml = jnp.where(row < _NL, avg, -1e30)
    m = jnp.max(ml, axis=1, keepdims=True)
    se = jnp.sum(jnp.exp(ml - m), axis=1, keepdims=True)
    lab = lab_ref[...]
    picked = jnp.sum(jnp.where(row == lab, avg, 0.0), axis=1, keepdims=True)
    nll = m + jnp.log(se) - picked
    avg_ref[...] = avg
    loss_ref[0, 0] = jnp.sum(nll) * (1.0 / (_B * _W))


def kernel(sent_logits, word_ids, labels, W_mlp):
    zeros_arr = jnp.zeros((_PAD, _W), jnp.float32)
    sums = []
    for ci in range(_CHUNK):
        probs_c = _tc1_call(sent_logits, W_mlp, ci)
        sums.append(_sc_seg_sum(probs_c, word_ids, zeros_arr, ci))

    avg, loss = pl.pallas_call(
        _tc2_body,
        in_specs=[
            pl.BlockSpec((_CB, _QP, _PAD, _W), lambda: (0, 0, 0, 0))
            for _ in range(_CHUNK)
        ] + [pl.BlockSpec((_B, 1, _W), lambda: (0, 0, 0))],
        out_specs=[
            pl.BlockSpec((_B, _PAD, _W), lambda: (0, 0, 0)),
            pl.BlockSpec((1, 1), lambda: (0, 0), memory_space=pltpu.SMEM),
        ],
        out_shape=[
            jax.ShapeDtypeStruct((_B, _PAD, _W), jnp.float32),
            jax.ShapeDtypeStruct((1, 1), jnp.float32),
        ],
    )(*sums, labels.reshape(_B, 1, _W))

    return jnp.swapaxes(avg[:, :_NL, :], 1, 2), loss[0, 0]
